# Initial kernel scaffold; baseline (speedup 1.0000x reference)
import jax
import jax.numpy as jnp
from jax.experimental import pallas as pl


def kernel(x, edge_index, snorm_n, snorm_e, adj_mask, W0, s0, W1, s1, W2, s2):
    raise NotImplementedError("write your pallas kernel here")


# speedup vs baseline: 2.6643x; 2.6643x over previous
import functools

import jax
import jax.numpy as jnp
from jax import lax
from jax.experimental import pallas as pl
from jax.experimental.pallas import tpu as pltpu
from jax.experimental.pallas import tpu_sc as plsc

N = 10000
E = 160000
F = 256
H = 512
C = 40

NC = 2
NS = 16
RCH = 80
NRC = N // RCH
KMAX = (NRC + NS - 1) // NS


def _mask_body(k, w_ref, s_ref, o_ref):
    s = s_ref[...]
    bits = lax.bitcast_convert_type(s, jnp.int32)

    def step(_, lohi):
        lo, hi = lohi
        mid = (lo + hi) // 2
        cnt = jnp.sum((bits <= mid).astype(jnp.int32))
        take = cnt >= k + 1
        return jnp.where(take, lo, mid + 1), jnp.where(take, mid, hi)

    lo, _ = lax.fori_loop(0, 31, step, (jnp.int32(0), jnp.int32(0x3F800000)))
    thr = lax.bitcast_convert_type(lo, jnp.float32)
    o_ref[...] = jnp.where(s < thr, 0.0, w_ref[...])


def _mask_weights(w, s, k):
    return pl.pallas_call(
        functools.partial(_mask_body, k),
        out_shape=jax.ShapeDtypeStruct(w.shape, jnp.float32),
    )(w, s)


def _make_sc_agg(ntab, width, with_deg, edge_split, ch):
    mesh = plsc.VectorSubcoreMesh(
        core_axis_name="c", subcore_axis_name="s", num_cores=NC,
        num_subcores=NS)
    ep = E // NS if not edge_split else E // (NS * NC)
    nch = ep // ch
    nout = ntab * (NC if edge_split else 1)

    def body(*refs):
        tabs = refs[:ntab]
        src_ref, dst_ref = refs[ntab], refs[ntab + 1]
        n_out_total = nout + (1 if with_deg else 0)
        outs = refs[ntab + 2: ntab + 2 + n_out_total]
        rest = refs[ntab + 2 + n_out_total:]
        (idx_s, idx_d, rows, zbuf, stg, acc, gsem) = rest[:7]
        if with_deg:
            ones_b, zd, stgd, dacc = rest[7:]
        c = lax.axis_index("c")
        sid = lax.axis_index("s")

        zero16 = jnp.zeros((16,), jnp.float32)

        def zinit(i, _):
            for j in range(width // 16):
                zbuf[i, pl.ds(16 * j, 16)] = zero16
            if with_deg:
                for j in range(4):
                    zd[i, pl.ds(16 * j, 16)] = zero16
            return 0

        lax.fori_loop(0, RCH, zinit, 0)

        def for_my_rows(fn):
            for k in range(KMAX):
                ci = sid + NS * k

                @pl.when(ci < NRC)
                def _():
                    fn(pl.multiple_of(ci * RCH, 8))

        def zero_acc():
            for_my_rows(lambda r: pltpu.sync_copy(zbuf, acc.at[pl.ds(r, RCH)]))

        zero_acc()

        if with_deg:
            one16 = jnp.ones((16,), jnp.float32)

            def oinit(i, _):
                for j in range(4):
                    ones_b[i, pl.ds(16 * j, 16)] = one16
                return 0

            lax.fori_loop(0, ch, oinit, 0)

            @pl.when(c == 0)
            def _():
                for_my_rows(
                    lambda r: pltpu.sync_copy(zd, dacc.at[pl.ds(r, RCH)]))

        plsc.subcore_barrier()

        def writeback(out_ref):
            def one(r):
                pltpu.sync_copy(acc.at[pl.ds(r, RCH)], stg)
                pltpu.sync_copy(stg, out_ref.at[pl.ds(r, RCH)])

            for_my_rows(one)

        for b in range(ntab):
            owner = b % NC

            def run_block(b=b, owner=owner):
                first_deg = with_deg and b == 0 and owner == 0

                def ebody(i, _):
                    if edge_split:
                        off = c * (E // NC) + sid * ep + i * ch
                    else:
                        off = sid * ep + i * ch
                    pltpu.sync_copy(src_ref.at[pl.ds(off, ch)], idx_s)
                    pltpu.sync_copy(dst_ref.at[pl.ds(off, ch)], idx_d)
                    pltpu.async_copy(tabs[b].at[idx_s], rows, gsem).wait()
                    pltpu.sync_copy(rows, acc.at[idx_d], add=True)
                    if first_deg:
                        pltpu.sync_copy(ones_b, dacc.at[idx_d], add=True)
                    return 0

                lax.fori_loop(0, nch, ebody, 0)
                plsc.subcore_barrier()
                if edge_split:
                    @pl.when(c == 0)
                    def _():
                        writeback(outs[2 * b])

                    @pl.when(c == 1)
                    def _():
                        writeback(outs[2 * b + 1])
                else:
                    writeback(outs[b])
                if first_deg:
                    @pl.when(c == 0)
                    def _():
                        def one(r):
                            pltpu.sync_copy(dacc.at[pl.ds(r, RCH)], stgd)
                            pltpu.sync_copy(stgd, outs[nout].at[pl.ds(r, RCH)])

                        for_my_rows(one)
                if b + NC < ntab:
                    zero_acc()
                    plsc.subcore_barrier()

            if edge_split:
                run_block()
            else:
                pl.when(c == owner)(run_block)

    out_type = [jax.ShapeDtypeStruct((N, width), jnp.float32)] * nout
    if with_deg:
        out_type = out_type + [jax.ShapeDtypeStruct((N, 64), jnp.float32)]
    scratch = [
        pltpu.VMEM((ch,), jnp.int32),
        pltpu.VMEM((ch,), jnp.int32),
        pltpu.VMEM((ch, width), jnp.float32),
        pltpu.VMEM((RCH, width), jnp.float32),
        pltpu.VMEM((RCH, width), jnp.float32),
        pltpu.VMEM_SHARED((N, width), jnp.float32),
        pltpu.SemaphoreType.DMA,
    ]
    if with_deg:
        scratch += [
            pltpu.VMEM((ch, 64), jnp.float32),
            pltpu.VMEM((RCH, 64), jnp.float32),
            pltpu.VMEM((RCH, 64), jnp.float32),
            pltpu.VMEM_SHARED((N, 64), jnp.float32),
        ]
    return pl.kernel(body, out_type=tuple(out_type), mesh=mesh,
                     scratch_types=tuple(scratch),
                     compiler_params=pltpu.CompilerParams(
                         use_tc_tiling_on_sc=False))


BN = 400
GRID = N // BN


def _inv_deg(deg_blk):
    return 1.0 / jnp.maximum(deg_blk[:, 0:1], 1.0)


def _h1z1_body(*refs):
    x_ref = refs[0]
    a_refs = refs[1:5]
    deg_ref, w0_ref, w1_ref = refs[5], refs[6], refs[7]
    o_refs = refs[8:16]
    inv = _inv_deg(deg_ref[...])
    agg = jnp.concatenate([a[...] for a in a_refs], axis=1)
    t = x_ref[...] + agg * inv
    h1 = jnp.maximum(
        lax.dot_general(t, w0_ref[...], (((1,), (1,)), ((), ())),
                        preferred_element_type=jnp.float32), 0.0)
    z = lax.dot_general(h1, w1_ref[...], (((1,), (1,)), ((), ())),
                        preferred_element_type=jnp.float32)
    for j in range(8):
        o_refs[j][...] = z[:, 64 * j:64 * (j + 1)]


def _h1z1(x, aggs, deg, w0m, w1m):
    blk = lambda w: pl.BlockSpec((BN, w), lambda i: (i, 0))
    full = lambda shp: pl.BlockSpec(shp, lambda i: (0, 0))
    return pl.pallas_call(
        _h1z1_body,
        grid=(GRID,),
        in_specs=[blk(F)] + [blk(64)] * 4 + [blk(64),
                  full((H, F)), full((H, H))],
        out_specs=[blk(64)] * 8,
        out_shape=[jax.ShapeDtypeStruct((N, 64), jnp.float32)] * 8,
    )(x, *aggs, deg, w0m, w1m)


def _h2z2_body(*refs):
    z_refs = refs[0:8]
    b_refs = refs[8:16]
    deg_ref, w2_ref, o_ref = refs[16], refs[17], refs[18]
    inv = _inv_deg(deg_ref[...])
    acc = jnp.zeros((BN, 64), jnp.float32)
    for b in range(8):
        h = jnp.maximum(z_refs[b][...] + b_refs[b][...] * inv, 0.0)
        acc = acc + lax.dot_general(
            h, w2_ref[:, 64 * b:64 * (b + 1)], (((1,), (1,)), ((), ())),
            preferred_element_type=jnp.float32)
    o_ref[...] = acc


def _h2z2(zs, bs, deg, w2m):
    blk = lambda w: pl.BlockSpec((BN, w), lambda i: (i, 0))
    return pl.pallas_call(
        _h2z2_body,
        grid=(GRID,),
        in_specs=[blk(64)] * 16 + [blk(64),
                  pl.BlockSpec((64, H), lambda i: (0, 0))],
        out_specs=blk(64),
        out_shape=jax.ShapeDtypeStruct((N, 64), jnp.float32),
    )(*zs, *bs, deg, w2m)


def _out_body(z2_ref, p0_ref, p1_ref, deg_ref, o_ref):
    inv = _inv_deg(deg_ref[...])
    t = z2_ref[...] + (p0_ref[...] + p1_ref[...]) * inv
    o_ref[...] = jnp.maximum(t, 0.0)[:, 0:C]


def _final(z2, p0, p1, deg):
    blk = lambda w: pl.BlockSpec((BN, w), lambda i: (i, 0))
    return pl.pallas_call(
        _out_body,
        grid=(GRID,),
        in_specs=[blk(64), blk(64), blk(64), blk(64)],
        out_specs=blk(C),
        out_shape=jax.ShapeDtypeStruct((N, C), jnp.float32),
    )(z2, p0, p1, deg)


_sc_agg0 = _make_sc_agg(ntab=4, width=64, with_deg=True, edge_split=False,
                        ch=80)
_sc_agg1 = _make_sc_agg(ntab=8, width=64, with_deg=False, edge_split=False,
                        ch=80)
_sc_agg2 = _make_sc_agg(ntab=1, width=64, with_deg=False, edge_split=True,
                        ch=40)


def kernel(x, edge_index, snorm_n, snorm_e, adj_mask, W0, s0, W1, s1, W2, s2):
    src = edge_index[0]
    dst = edge_index[1]

    w0m = _mask_weights(W0, s0, (H * F) // 2)
    w1m = _mask_weights(W1, s1, (H * H) // 2)
    w2p = jnp.pad(W2, ((0, 64 - C), (0, 0)))
    s2p = jnp.pad(s2, ((0, 64 - C), (0, 0)), constant_values=2.0)
    w2m = _mask_weights(w2p, s2p, (C * H) // 2)

    xblocks = [x[:, 64 * j:64 * (j + 1)] for j in range(4)]
    *a_blocks, deg = _sc_agg0(*xblocks, src, dst)

    zs = _h1z1(x, a_blocks, deg, w0m, w1m)
    bs = _sc_agg1(*zs, src, dst)
    z2 = _h2z2(zs, bs, deg, w2m)
    p0, p1 = _sc_agg2(z2, src, dst)
    return _final(z2, p0, p1, deg)


# speedup vs baseline: 3.5393x; 1.3284x over previous
import functools

import jax
import jax.numpy as jnp
from jax import lax
from jax.experimental import pallas as pl
from jax.experimental.pallas import tpu as pltpu
from jax.experimental.pallas import tpu_sc as plsc

N = 10000
E = 160000
F = 256
H = 512
C = 40

NC = 2
NS = 16
CH = 128
EROWS = 1280
E2 = EROWS * CH
SINK = N
NP = N + 16
RPT = EROWS // NS
RCH = 80
NRC = N // RCH
KMAX = (NRC + NS - 1) // NS


def _mask_body(k, w_ref, s_ref, o_ref):
    s = s_ref[...]
    bits = lax.bitcast_convert_type(s, jnp.int32)

    def step(_, lohi):
        lo, hi = lohi
        mid = (lo + hi) // 2
        cnt = jnp.sum((bits <= mid).astype(jnp.int32))
        take = cnt >= k + 1
        return jnp.where(take, lo, mid + 1), jnp.where(take, mid, hi)

    lo, _ = lax.fori_loop(0, 31, step, (jnp.int32(0), jnp.int32(0x3F800000)))
    thr = lax.bitcast_convert_type(lo, jnp.float32)
    o_ref[...] = jnp.where(s < thr, 0.0, w_ref[...])


def _mask_weights(w, s, k):
    return pl.pallas_call(
        functools.partial(_mask_body, k),
        out_shape=jax.ShapeDtypeStruct(w.shape, jnp.float32),
    )(w, s)


def _make_sc_agg(ntab, with_deg, edge_split):
    mesh = plsc.VectorSubcoreMesh(
        core_axis_name="c", subcore_axis_name="s", num_cores=NC,
        num_subcores=NS)
    nout = ntab * (NC if edge_split else 1)
    n_out_total = nout + (2 if with_deg else 0)
    rpt = RPT if not edge_split else RPT // NC

    def body(*refs):
        tabs = refs[:ntab]
        src_ref, dst_ref, zrows = refs[ntab], refs[ntab + 1], refs[ntab + 2]
        outs = refs[ntab + 3: ntab + 3 + n_out_total]
        (idx_s, idx_d, rows0, rows1, acc, dacc,
         gs0, gs1, ss0, ss1, bulk) = refs[ntab + 3 + n_out_total:]
        c = lax.axis_index("c")
        sid = lax.axis_index("s")

        def for_my_rows(fn):
            for k in range(KMAX):
                ci = sid + NS * k

                @pl.when(ci < NRC)
                def _():
                    fn(pl.multiple_of(ci * RCH, 8))

        def fire_zero(a):
            for_my_rows(lambda r: pltpu.async_copy(
                zrows, a.at[pl.ds(r, RCH)], bulk))

        def drain_zero(a):
            for_my_rows(lambda r: pltpu.make_async_copy(
                zrows, a.at[pl.ds(r, RCH)], bulk).wait())

        def fire_wb(a, out_ref):
            for_my_rows(lambda r: pltpu.async_copy(
                a.at[pl.ds(r, RCH)], out_ref.at[pl.ds(r, RCH)], bulk))

        def drain_wb(a, out_ref):
            for_my_rows(lambda r: pltpu.make_async_copy(
                a.at[pl.ds(r, RCH)], out_ref.at[pl.ds(r, RCH)], bulk).wait())

        fire_zero(acc)
        if with_deg:
            fire_zero(dacc)

        @pl.when(sid == 0)
        def _():
            pltpu.sync_copy(zrows.at[pl.ds(0, 16)], acc.at[pl.ds(N, 16)])
            if with_deg:
                pltpu.sync_copy(zrows.at[pl.ds(0, 16)], dacc.at[pl.ds(N, 16)])

        drain_zero(acc)
        if with_deg:
            drain_zero(dacc)
        plsc.subcore_barrier()

        def gather(tab, a, rbuf, sem):
            pltpu.async_copy(tab.at[idx_s.at[a]], rbuf, sem)

        def wait_gather(tab, rbuf, sem):
            pltpu.make_async_copy(tab.at[idx_s.at[0]], rbuf, sem).wait()

        def scatter(rbuf, a, sem):
            pltpu.async_copy(rbuf, acc.at[idx_d.at[a]], sem, add=True)

        def wait_scatter(rbuf, sem):
            pltpu.make_async_copy(rbuf, acc.at[idx_d.at[0]], sem).wait()

        def run_block(b):
            if edge_split:
                r0 = c * (EROWS // NC) + sid * rpt
            else:
                r0 = sid * rpt
            pltpu.sync_copy(src_ref.at[pl.ds(r0, rpt)],
                            idx_s.at[pl.ds(0, rpt)])
            pltpu.sync_copy(dst_ref.at[pl.ds(r0, rpt)],
                            idx_d.at[pl.ds(0, rpt)])
            tab = tabs[b]
            gather(tab, 0, rows0, gs0)
            gather(tab, 1, rows1, gs1)

            def pipe(j, _):
                a = 2 * j
                wait_gather(tab, rows0, gs0)
                scatter(rows0, a, ss0)
                wait_gather(tab, rows1, gs1)
                scatter(rows1, a + 1, ss1)
                wait_scatter(rows0, ss0)
                gather(tab, a + 2, rows0, gs0)
                wait_scatter(rows1, ss1)
                gather(tab, a + 3, rows1, gs1)
                return 0

            lax.fori_loop(0, rpt // 2 - 1, pipe, 0)
            a = rpt - 2
            wait_gather(tab, rows0, gs0)
            scatter(rows0, a, ss0)
            wait_gather(tab, rows1, gs1)
            scatter(rows1, a + 1, ss1)
            wait_scatter(rows0, ss0)
            wait_scatter(rows1, ss1)
            plsc.subcore_barrier()
            if edge_split:
                @pl.when(c == 0)
                def _():
                    fire_wb(acc, outs[2 * b])
                    drain_wb(acc, outs[2 * b])

                @pl.when(c == 1)
                def _():
                    fire_wb(acc, outs[2 * b + 1])
                    drain_wb(acc, outs[2 * b + 1])
            else:
                fire_wb(acc, outs[b])
                drain_wb(acc, outs[b])
            if b + NC < ntab:
                fire_zero(acc)
                drain_zero(acc)
                plsc.subcore_barrier()

        for b in range(ntab):
            if edge_split:
                run_block(b)
            else:
                pl.when(c == b % NC)(functools.partial(run_block, b))

        if with_deg:
            one16 = jnp.ones((16,), jnp.float32)

            def oinit(i, _):
                for j in range(4):
                    rows0[i, pl.ds(16 * j, 16)] = one16
                return 0

            lax.fori_loop(0, CH, oinit, 0)
            drpt = RPT // NC
            r0 = c * (EROWS // NC) + sid * drpt
            pltpu.sync_copy(dst_ref.at[pl.ds(r0, drpt)],
                            idx_d.at[pl.ds(0, drpt)])
            for k in range(drpt):
                pltpu.async_copy(rows0, dacc.at[idx_d.at[k]], bulk, add=True)
            for k in range(drpt):
                pltpu.make_async_copy(rows0, dacc.at[idx_d.at[0]],
                                      bulk).wait()
            plsc.subcore_barrier()

            @pl.when(c == 0)
            def _():
                fire_wb(dacc, outs[nout])
                drain_wb(dacc, outs[nout])

            @pl.when(c == 1)
            def _():
                fire_wb(dacc, outs[nout + 1])
                drain_wb(dacc, outs[nout + 1])

    out_type = [jax.ShapeDtypeStruct((N, 64), jnp.float32)] * n_out_total
    scratch = [
        pltpu.VMEM((RPT, CH), jnp.int32),
        pltpu.VMEM((RPT, CH), jnp.int32),
        pltpu.VMEM((CH, 64), jnp.float32),
        pltpu.VMEM((CH, 64), jnp.float32),
        pltpu.VMEM_SHARED((NP, 64), jnp.float32),
        pltpu.VMEM_SHARED((NP, 64), jnp.float32),
        pltpu.SemaphoreType.DMA,
        pltpu.SemaphoreType.DMA,
        pltpu.SemaphoreType.DMA,
        pltpu.SemaphoreType.DMA,
        pltpu.SemaphoreType.DMA,
    ]
    return pl.kernel(body, out_type=tuple(out_type), mesh=mesh,
                     scratch_types=tuple(scratch),
                     compiler_params=pltpu.CompilerParams(
                         use_tc_tiling_on_sc=False))


BN = 400
GRID = N // BN


def _inv_deg(d0, d1):
    return 1.0 / jnp.maximum(d0[:, 0:1] + d1[:, 0:1], 1.0)


def _h1z1_body(*refs):
    x_ref = refs[0]
    a_refs = refs[1:5]
    d0_ref, d1_ref, w0_ref, w1_ref = refs[5], refs[6], refs[7], refs[8]
    o_refs = refs[9:17]
    inv = _inv_deg(d0_ref[...], d1_ref[...])
    agg = jnp.concatenate([a[...] for a in a_refs], axis=1)
    t = x_ref[...] + agg * inv
    h1 = jnp.maximum(
        lax.dot_general(t, w0_ref[...], (((1,), (1,)), ((), ())),
                        preferred_element_type=jnp.float32), 0.0)
    z = lax.dot_general(h1, w1_ref[...], (((1,), (1,)), ((), ())),
                        preferred_element_type=jnp.float32)
    for j in range(8):
        o_refs[j][...] = z[:, 64 * j:64 * (j + 1)]


def _h1z1(x, aggs, d0, d1, w0m, w1m):
    blk = lambda w: pl.BlockSpec((BN, w), lambda i: (i, 0))
    full = lambda shp: pl.BlockSpec(shp, lambda i: (0, 0))
    return pl.pallas_call(
        _h1z1_body,
        grid=(GRID,),
        in_specs=[blk(F)] + [blk(64)] * 4 + [blk(64), blk(64),
                  full((H, F)), full((H, H))],
        out_specs=[blk(64)] * 8,
        out_shape=[jax.ShapeDtypeStruct((N, 64), jnp.float32)] * 8,
    )(x, *aggs, d0, d1, w0m, w1m)


def _h2z2_body(*refs):
    z_refs = refs[0:8]
    b_refs = refs[8:16]
    d0_ref, d1_ref, w2_ref, o_ref = refs[16], refs[17], refs[18], refs[19]
    inv = _inv_deg(d0_ref[...], d1_ref[...])
    acc = jnp.zeros((BN, 64), jnp.float32)
    for b in range(8):
        h = jnp.maximum(z_refs[b][...] + b_refs[b][...] * inv, 0.0)
        acc = acc + lax.dot_general(
            h, w2_ref[:, 64 * b:64 * (b + 1)], (((1,), (1,)), ((), ())),
            preferred_element_type=jnp.float32)
    o_ref[...] = acc


def _h2z2(zs, bs, d0, d1, w2m):
    blk = lambda w: pl.BlockSpec((BN, w), lambda i: (i, 0))
    return pl.pallas_call(
        _h2z2_body,
        grid=(GRID,),
        in_specs=[blk(64)] * 16 + [blk(64), blk(64),
                  pl.BlockSpec((64, H), lambda i: (0, 0))],
        out_specs=blk(64),
        out_shape=jax.ShapeDtypeStruct((N, 64), jnp.float32),
    )(*zs, *bs, d0, d1, w2m)


def _out_body(z2_ref, p0_ref, p1_ref, d0_ref, d1_ref, o_ref):
    inv = _inv_deg(d0_ref[...], d1_ref[...])
    t = z2_ref[...] + (p0_ref[...] + p1_ref[...]) * inv
    o_ref[...] = jnp.maximum(t, 0.0)[:, 0:C]


def _final(z2, p0, p1, d0, d1):
    blk = lambda w: pl.BlockSpec((BN, w), lambda i: (i, 0))
    return pl.pallas_call(
        _out_body,
        grid=(GRID,),
        in_specs=[blk(64)] * 5,
        out_specs=blk(C),
        out_shape=jax.ShapeDtypeStruct((N, C), jnp.float32),
    )(z2, p0, p1, d0, d1)


_sc_agg0 = _make_sc_agg(ntab=4, with_deg=True, edge_split=False)
_sc_agg1 = _make_sc_agg(ntab=8, with_deg=False, edge_split=False)
_sc_agg2 = _make_sc_agg(ntab=1, with_deg=False, edge_split=True)


def kernel(x, edge_index, snorm_n, snorm_e, adj_mask, W0, s0, W1, s1, W2, s2):
    src = edge_index[0]
    dst = edge_index[1]
    pad = E2 - E
    src2 = jnp.concatenate([src, jnp.zeros((pad,), jnp.int32)]
                           ).reshape(EROWS, CH)
    dst2 = jnp.concatenate([dst, jnp.full((pad,), SINK, jnp.int32)]
                           ).reshape(EROWS, CH)
    zrows = jnp.zeros((RCH, 64), jnp.float32)

    w0m = _mask_weights(W0, s0, (H * F) // 2)
    w1m = _mask_weights(W1, s1, (H * H) // 2)
    w2p = jnp.pad(W2, ((0, 64 - C), (0, 0)))
    s2p = jnp.pad(s2, ((0, 64 - C), (0, 0)), constant_values=2.0)
    w2m = _mask_weights(w2p, s2p, (C * H) // 2)

    xblocks = [x[:, 64 * j:64 * (j + 1)] for j in range(4)]
    *a_blocks, deg0, deg1 = _sc_agg0(*xblocks, src2, dst2, zrows)

    zs = _h1z1(x, a_blocks, deg0, deg1, w0m, w1m)
    bs = _sc_agg1(*zs, src2, dst2, zrows)
    z2 = _h2z2(zs, bs, deg0, deg1, w2m)
    p0, p1 = _sc_agg2(z2, src2, dst2, zrows)
    return _final(z2, p0, p1, deg0, deg1)


# speedup vs baseline: 3.9706x; 1.1218x over previous
import functools

import jax
import jax.numpy as jnp
from jax import lax
from jax.experimental import pallas as pl
from jax.experimental.pallas import tpu as pltpu
from jax.experimental.pallas import tpu_sc as plsc

N = 10000
E = 160000
F = 256
H = 512
C = 40
W = 128

NC = 2
NS = 16
CH = 128
EROWS = 1280
E2 = EROWS * CH
SINK = N
NP = N + 16
RPT = EROWS // NS
HPT = RPT // 2
RCH = 80
NRC = N // RCH
KMAX = (NRC + NS - 1) // NS


def _mask_body(k, w_ref, s_ref, o_ref):
    s = s_ref[...]
    bits = lax.bitcast_convert_type(s, jnp.int32)

    def step(_, lohi):
        lo, hi = lohi
        mid = (lo + hi) // 2
        cnt = jnp.sum((bits <= mid).astype(jnp.int32))
        take = cnt >= k + 1
        return jnp.where(take, lo, mid + 1), jnp.where(take, mid, hi)

    lo, _ = lax.fori_loop(0, 31, step, (jnp.int32(0), jnp.int32(0x3F800000)))
    thr = lax.bitcast_convert_type(lo, jnp.float32)
    o_ref[...] = jnp.where(s < thr, 0.0, w_ref[...])


def _mask_weights(w, s, k):
    return pl.pallas_call(
        functools.partial(_mask_body, k),
        out_shape=jax.ShapeDtypeStruct(w.shape, jnp.float32),
    )(w, s)


def _make_sc_agg(ntab, with_deg, edge_split):
    mesh = plsc.VectorSubcoreMesh(
        core_axis_name="c", subcore_axis_name="s", num_cores=NC,
        num_subcores=NS)
    nout = ntab * (NC if edge_split else 1)
    n_out_total = nout + (2 if with_deg else 0)

    def body(*refs):
        tabs = refs[:ntab]
        src_ref, dst_ref, zrows = refs[ntab], refs[ntab + 1], refs[ntab + 2]
        outs = refs[ntab + 3: ntab + 3 + n_out_total]
        (idx_s, idx_d, rows0, rows1,
         acc, gs0, gs1, ss0, ss1, bulk) = refs[ntab + 3 + n_out_total:]
        c = lax.axis_index("c")
        sid = lax.axis_index("s")

        def for_my_rows(fn):
            for k in range(KMAX):
                ci = sid + NS * k

                @pl.when(ci < NRC)
                def _():
                    fn(pl.multiple_of(ci * RCH, 8))

        def fire_zero():
            for_my_rows(lambda r: pltpu.async_copy(
                zrows, acc.at[pl.ds(r, RCH)], bulk))

        def drain_zero():
            for_my_rows(lambda r: pltpu.make_async_copy(
                zrows, acc.at[pl.ds(r, RCH)], bulk).wait())

        def fire_wb(out_ref):
            for_my_rows(lambda r: pltpu.async_copy(
                acc.at[pl.ds(r, RCH)], out_ref.at[pl.ds(r, RCH)], bulk))

        def drain_wb(out_ref):
            for_my_rows(lambda r: pltpu.make_async_copy(
                acc.at[pl.ds(r, RCH)], out_ref.at[pl.ds(r, RCH)], bulk).wait())

        def zero_sink():
            @pl.when(sid == 0)
            def _():
                pltpu.sync_copy(zrows.at[pl.ds(0, 16)], acc.at[pl.ds(N, 16)])

        fire_zero()
        zero_sink()
        drain_zero()
        plsc.subcore_barrier()

        def gather(tab, a, rbuf, sem):
            pltpu.async_copy(tab.at[idx_s.at[a]], rbuf, sem)

        def wait_gather(tab, rbuf, sem):
            pltpu.make_async_copy(tab.at[idx_s.at[0]], rbuf, sem).wait()

        def scatter(rbuf, a, sem):
            pltpu.async_copy(rbuf, acc.at[idx_d.at[a]], sem, add=True)

        def wait_scatter(rbuf, sem):
            pltpu.make_async_copy(rbuf, acc.at[idx_d.at[0]], sem).wait()

        def run_half(tab, r0):
            pltpu.sync_copy(src_ref.at[pl.ds(r0, HPT)],
                            idx_s.at[pl.ds(0, HPT)])
            pltpu.sync_copy(dst_ref.at[pl.ds(r0, HPT)],
                            idx_d.at[pl.ds(0, HPT)])
            gather(tab, 0, rows0, gs0)
            gather(tab, 1, rows1, gs1)

            def pipe(j, _):
                a = 2 * j
                wait_gather(tab, rows0, gs0)
                scatter(rows0, a, ss0)
                wait_gather(tab, rows1, gs1)
                scatter(rows1, a + 1, ss1)
                wait_scatter(rows0, ss0)
                gather(tab, a + 2, rows0, gs0)
                wait_scatter(rows1, ss1)
                gather(tab, a + 3, rows1, gs1)
                return 0

            lax.fori_loop(0, HPT // 2 - 1, pipe, 0)
            a = HPT - 2
            wait_gather(tab, rows0, gs0)
            scatter(rows0, a, ss0)
            wait_gather(tab, rows1, gs1)
            scatter(rows1, a + 1, ss1)
            wait_scatter(rows0, ss0)
            wait_scatter(rows1, ss1)

        def run_block(b):
            if edge_split:
                base = c * (EROWS // NC) + sid * (RPT // NC)
                run_half(tabs[b], base)
            else:
                run_half(tabs[b], sid * RPT)
                run_half(tabs[b], sid * RPT + HPT)
            plsc.subcore_barrier()
            if edge_split:
                @pl.when(c == 0)
                def _():
                    fire_wb(outs[2 * b])
                    drain_wb(outs[2 * b])

                @pl.when(c == 1)
                def _():
                    fire_wb(outs[2 * b + 1])
                    drain_wb(outs[2 * b + 1])
            else:
                fire_wb(outs[b])
                drain_wb(outs[b])
            if b + NC < ntab or with_deg:
                fire_zero()
                zero_sink()
                drain_zero()
                plsc.subcore_barrier()

        for b in range(ntab):
            if edge_split:
                run_block(b)
            else:
                pl.when(c == b % NC)(functools.partial(run_block, b))

        if with_deg:
            one16 = jnp.ones((16,), jnp.float32)

            def oinit(i, _):
                for j in range(W // 16):
                    rows0[i, pl.ds(16 * j, 16)] = one16
                return 0

            lax.fori_loop(0, CH, oinit, 0)
            drpt = RPT // NC
            r0 = c * (EROWS // NC) + sid * drpt
            pltpu.sync_copy(dst_ref.at[pl.ds(r0, drpt)],
                            idx_d.at[pl.ds(0, drpt)])
            for k in range(drpt):
                pltpu.async_copy(rows0, acc.at[idx_d.at[k]], bulk, add=True)
            for k in range(drpt):
                pltpu.make_async_copy(rows0, acc.at[idx_d.at[0]],
                                      bulk).wait()
            plsc.subcore_barrier()

            @pl.when(c == 0)
            def _():
                fire_wb(outs[nout])
                drain_wb(outs[nout])

            @pl.when(c == 1)
            def _():
                fire_wb(outs[nout + 1])
                drain_wb(outs[nout + 1])

    out_type = [jax.ShapeDtypeStruct((N, W), jnp.float32)] * n_out_total
    scratch = [
        pltpu.VMEM((HPT, CH), jnp.int32),
        pltpu.VMEM((HPT, CH), jnp.int32),
        pltpu.VMEM((CH, W), jnp.float32),
        pltpu.VMEM((CH, W), jnp.float32),
        pltpu.VMEM_SHARED((NP, W), jnp.float32),
        pltpu.SemaphoreType.DMA,
        pltpu.SemaphoreType.DMA,
        pltpu.SemaphoreType.DMA,
        pltpu.SemaphoreType.DMA,
        pltpu.SemaphoreType.DMA,
    ]
    return pl.kernel(body, out_type=tuple(out_type), mesh=mesh,
                     scratch_types=tuple(scratch),
                     compiler_params=pltpu.CompilerParams(
                         use_tc_tiling_on_sc=False))


BN = 400
GRID = N // BN


def _inv_deg(d0, d1):
    return 1.0 / jnp.maximum(d0[:, 0:1] + d1[:, 0:1], 1.0)


def _h1z1_body(*refs):
    x_ref = refs[0]
    a_refs = refs[1:3]
    d0_ref, d1_ref, w0_ref, w1_ref = refs[3], refs[4], refs[5], refs[6]
    o_refs = refs[7:11]
    inv = _inv_deg(d0_ref[...], d1_ref[...])
    agg = jnp.concatenate([a[...] for a in a_refs], axis=1)
    t = x_ref[...] + agg * inv
    h1 = jnp.maximum(
        lax.dot_general(t, w0_ref[...], (((1,), (1,)), ((), ())),
                        preferred_element_type=jnp.float32), 0.0)
    z = lax.dot_general(h1, w1_ref[...], (((1,), (1,)), ((), ())),
                        preferred_element_type=jnp.float32)
    for j in range(4):
        o_refs[j][...] = z[:, W * j:W * (j + 1)]


def _h1z1(x, aggs, d0, d1, w0m, w1m):
    blk = lambda w: pl.BlockSpec((BN, w), lambda i: (i, 0))
    full = lambda shp: pl.BlockSpec(shp, lambda i: (0, 0))
    return pl.pallas_call(
        _h1z1_body,
        grid=(GRID,),
        in_specs=[blk(F)] + [blk(W)] * 2 + [blk(W), blk(W),
                  full((H, F)), full((H, H))],
        out_specs=[blk(W)] * 4,
        out_shape=[jax.ShapeDtypeStruct((N, W), jnp.float32)] * 4,
    )(x, *aggs, d0, d1, w0m, w1m)


def _h2z2_body(*refs):
    z_refs = refs[0:4]
    b_refs = refs[4:8]
    d0_ref, d1_ref, w2_ref, o_ref = refs[8], refs[9], refs[10], refs[11]
    inv = _inv_deg(d0_ref[...], d1_ref[...])
    acc = jnp.zeros((BN, W), jnp.float32)
    for b in range(4):
        h = jnp.maximum(z_refs[b][...] + b_refs[b][...] * inv, 0.0)
        acc = acc + lax.dot_general(
            h, w2_ref[:, W * b:W * (b + 1)], (((1,), (1,)), ((), ())),
            preferred_element_type=jnp.float32)
    o_ref[...] = acc


def _h2z2(zs, bs, d0, d1, w2m):
    blk = lambda w: pl.BlockSpec((BN, w), lambda i: (i, 0))
    return pl.pallas_call(
        _h2z2_body,
        grid=(GRID,),
        in_specs=[blk(W)] * 8 + [blk(W), blk(W),
                  pl.BlockSpec((W, H), lambda i: (0, 0))],
        out_specs=blk(W),
        out_shape=jax.ShapeDtypeStruct((N, W), jnp.float32),
    )(*zs, *bs, d0, d1, w2m)


def _out_body(z2_ref, p0_ref, p1_ref, d0_ref, d1_ref, o_ref):
    inv = _inv_deg(d0_ref[...], d1_ref[...])
    t = z2_ref[...] + (p0_ref[...] + p1_ref[...]) * inv
    o_ref[...] = jnp.maximum(t, 0.0)[:, 0:C]


def _final(z2, p0, p1, d0, d1):
    blk = lambda w: pl.BlockSpec((BN, w), lambda i: (i, 0))
    return pl.pallas_call(
        _out_body,
        grid=(GRID,),
        in_specs=[blk(W)] * 5,
        out_specs=blk(C),
        out_shape=jax.ShapeDtypeStruct((N, C), jnp.float32),
    )(z2, p0, p1, d0, d1)


_sc_agg0 = _make_sc_agg(ntab=2, with_deg=True, edge_split=False)
_sc_agg1 = _make_sc_agg(ntab=4, with_deg=False, edge_split=False)
_sc_agg2 = _make_sc_agg(ntab=1, with_deg=False, edge_split=True)


def kernel(x, edge_index, snorm_n, snorm_e, adj_mask, W0, s0, W1, s1, W2, s2):
    src = edge_index[0]
    dst = edge_index[1]
    pad = E2 - E
    src2 = jnp.concatenate([src, jnp.zeros((pad,), jnp.int32)]
                           ).reshape(EROWS, CH)
    dst2 = jnp.concatenate([dst, jnp.full((pad,), SINK, jnp.int32)]
                           ).reshape(EROWS, CH)
    zrows = jnp.zeros((RCH, W), jnp.float32)

    w0m = _mask_weights(W0, s0, (H * F) // 2)
    w1m = _mask_weights(W1, s1, (H * H) // 2)
    w2p = jnp.pad(W2, ((0, W - C), (0, 0)))
    s2p = jnp.pad(s2, ((0, W - C), (0, 0)), constant_values=2.0)
    w2m = _mask_weights(w2p, s2p, (C * H) // 2)

    xblocks = [x[:, W * j:W * (j + 1)] for j in range(2)]
    a0, a1, deg0, deg1 = _sc_agg0(*xblocks, src2, dst2, zrows)

    zs = _h1z1(x, [a0, a1], deg0, deg1, w0m, w1m)
    bs = _sc_agg1(*zs, src2, dst2, zrows)
    z2 = _h2z2(zs, bs, deg0, deg1, w2m)
    p0, p1 = _sc_agg2(z2, src2, dst2, zrows)
    return _final(z2, p0, p1, deg0, deg1)
